# Initial kernel scaffold; baseline (speedup 1.0000x reference)
import jax
import jax.numpy as jnp
from jax.experimental import pallas as pl


def kernel(edge_index, etypes, norm, emb, basis1, comp1, bias1, basis2, comp2, bias2):
    raise NotImplementedError("write your pallas kernel here")


# speedup vs baseline: 6.5053x; 6.5053x over previous
import functools

import jax
import jax.numpy as jnp
from jax import lax
from jax.experimental import pallas as pl
from jax.experimental.pallas import tpu as pltpu
from jax.experimental.pallas import tpu_sc as plsc

_NC = 2
_NS = 16
_L = 16
_NW = _NC * _NS

_CH = 128


def _project_body(x_ref, basis_ref, comp_ref, y_ref, *, R, B):
    x = x_ref[...]
    z = [jnp.dot(x, basis_ref[b], preferred_element_type=jnp.float32)
         for b in range(B)]
    for r in range(R):
        acc = z[0] * comp_ref[r, 0]
        for b in range(1, B):
            acc = acc + z[b] * comp_ref[r, b]
        y_ref[:, r, :] = acc


def _combine_project_body(p_ref, bias_ref, basis_ref, comp_ref, y_ref, *, R, B):
    h = jax.nn.relu(p_ref[0] + p_ref[1] + bias_ref[...])
    z = [jnp.dot(h, basis_ref[b], preferred_element_type=jnp.float32)
         for b in range(B)]
    for r in range(R):
        acc = z[0] * comp_ref[r, 0]
        for b in range(1, B):
            acc = acc + z[b] * comp_ref[r, b]
        y_ref[:, r, :] = acc


def _final_body(p_ref, bias_ref, o_ref):
    o_ref[...] = p_ref[0] + p_ref[1] + bias_ref[...]


def _project(x, basis, comp, bn):
    n, h = x.shape
    b, _, o = basis.shape
    r = comp.shape[0]
    return pl.pallas_call(
        functools.partial(_project_body, R=r, B=b),
        grid=(n // bn,),
        in_specs=[
            pl.BlockSpec((bn, h), lambda i: (i, 0)),
            pl.BlockSpec((b, h, o), lambda i: (0, 0, 0)),
            pl.BlockSpec(memory_space=pltpu.SMEM),
        ],
        out_specs=pl.BlockSpec((bn, r, o), lambda i: (i, 0, 0)),
        out_shape=jax.ShapeDtypeStruct((n, r, o), jnp.float32),
    )(x, basis, comp)


def _combine_project(p, bias, basis, comp, bn):
    _, n, h = p.shape
    b, _, o = basis.shape
    r = comp.shape[0]
    return pl.pallas_call(
        functools.partial(_combine_project_body, R=r, B=b),
        grid=(n // bn,),
        in_specs=[
            pl.BlockSpec((2, bn, h), lambda i: (0, i, 0)),
            pl.BlockSpec((1, h), lambda i: (0, 0)),
            pl.BlockSpec((b, h, o), lambda i: (0, 0, 0)),
            pl.BlockSpec(memory_space=pltpu.SMEM),
        ],
        out_specs=pl.BlockSpec((bn, r, o), lambda i: (i, 0, 0)),
        out_shape=jax.ShapeDtypeStruct((n, r, o), jnp.float32),
    )(p, bias.reshape(1, h), basis, comp)


def _final(p, bias, bn):
    _, n, h = p.shape
    return pl.pallas_call(
        _final_body,
        grid=(n // bn,),
        in_specs=[
            pl.BlockSpec((2, bn, h), lambda i: (0, i, 0)),
            pl.BlockSpec((1, h), lambda i: (0, 0)),
        ],
        out_specs=pl.BlockSpec((bn, h), lambda i: (i, 0)),
        out_shape=jax.ShapeDtypeStruct((n, h), jnp.float32),
    )(p, bias.reshape(1, h))


def _make_edge_agg(n_nodes, feat, e_pad, rel):
    per_w = e_pad // _NW
    n_chunks = per_w // _CH
    rows_per_tile = n_nodes // _NS
    assert rows_per_tile % 8 == 0 and rows_per_tile * _NS == n_nodes
    mesh = plsc.VectorSubcoreMesh(core_axis_name="c", subcore_axis_name="s")

    @functools.partial(
        pl.kernel,
        mesh=mesh,
        out_type=jax.ShapeDtypeStruct((_NC, n_nodes, feat), jnp.float32),
        scratch_types=[
            pltpu.VMEM((_CH,), jnp.int32),
            pltpu.VMEM((_CH,), jnp.int32),
            pltpu.VMEM((_CH,), jnp.int32),
            pltpu.VMEM((_CH, _L), jnp.float32),
            pltpu.VMEM((_CH,), jnp.int32),
            pltpu.VMEM((_CH, feat), jnp.float32),
            pltpu.VMEM((_CH, feat), jnp.float32),
            pltpu.VMEM_SHARED((n_nodes, feat), jnp.float32),
            pltpu.SemaphoreType.DMA,
        ],
    )
    def edge_agg(table, srcs, ets, dsts, norms, zrows, out,
                 src_v, et_v, dst_v, norm_v, gidx_v, rows_v, zb_v, acc, sem):
        cid = lax.axis_index("c")
        sid = lax.axis_index("s")
        wid = cid * _NS + sid
        ebase = wid * per_w
        r0 = sid * rows_per_tile

        row_chunks = []
        off = 0
        while off < rows_per_tile:
            sz = min(_CH, rows_per_tile - off)
            row_chunks.append((off, sz))
            off += sz

        for off, sz in row_chunks:
            pltpu.sync_copy(zrows.at[pl.ds(r0 + off, sz)], zb_v.at[pl.ds(0, sz)])
            pltpu.sync_copy(zb_v.at[pl.ds(0, sz)], acc.at[pl.ds(r0 + off, sz)])
        plsc.subcore_barrier()

        def chunk_body(ci, carry):
            base = ebase + ci * _CH
            c1 = pltpu.async_copy(srcs.at[pl.ds(base, _CH)], src_v, sem)
            c2 = pltpu.async_copy(ets.at[pl.ds(base, _CH)], et_v, sem)
            c3 = pltpu.async_copy(dsts.at[pl.ds(base, _CH)], dst_v, sem)
            c4 = pltpu.async_copy(norms.at[pl.ds(base, _CH)], norm_v, sem)
            c1.wait()
            c2.wait()
            c3.wait()
            c4.wait()
            for g in range(_CH // _L):
                sl = pl.ds(g * _L, _L)
                gidx_v[sl] = src_v[sl] * rel + et_v[sl]
            pltpu.async_copy(table.at[gidx_v], rows_v, sem).wait()

            def scale_body(j, c2_):
                nb = norm_v[j, :]
                for g2 in range(feat // _L):
                    s2 = pl.ds(g2 * _L, _L)
                    rows_v[j, s2] = rows_v[j, s2] * nb
                return c2_

            lax.fori_loop(0, _CH, scale_body, 0)
            pltpu.sync_copy(rows_v, acc.at[dst_v], add=True)
            return carry

        lax.fori_loop(0, n_chunks, chunk_body, 0)

        plsc.subcore_barrier()
        for off, sz in row_chunks:
            pltpu.sync_copy(acc.at[pl.ds(r0 + off, sz)], zb_v.at[pl.ds(0, sz)])
            pltpu.sync_copy(zb_v.at[pl.ds(0, sz)], out.at[cid, pl.ds(r0 + off, sz)])

    return edge_agg


def kernel(edge_index, etypes, norm, emb, basis1, comp1, bias1,
           basis2, comp2, bias2):
    n, h = emb.shape
    out_dim = basis2.shape[2]
    rel = comp1.shape[0]
    e = etypes.shape[0]

    src = edge_index[0].astype(jnp.int32)
    dst = edge_index[1].astype(jnp.int32)
    et = etypes.astype(jnp.int32)
    nrm = norm.reshape(e).astype(jnp.float32)

    e_pad = ((e + _NW * _CH - 1) // (_NW * _CH)) * (_NW * _CH)
    pad = e_pad - e
    src_p = jnp.pad(src, (0, pad))
    dst_p = jnp.pad(dst, (0, pad))
    et_p = jnp.pad(et, (0, pad))
    nrm_p = jnp.pad(nrm, (0, pad))
    nrm16 = jnp.broadcast_to(nrm_p[:, None], (e_pad, _L))

    n_pad = ((n + _NS * 8 - 1) // (_NS * 8)) * (_NS * 8)
    zeros_nh = jnp.zeros((n_pad, h), jnp.float32)
    zeros_no = jnp.zeros((n_pad, out_dim), jnp.float32)

    bn = 1000
    agg1 = _make_edge_agg(n_pad, h, e_pad, rel)
    agg2 = _make_edge_agg(n_pad, out_dim, e_pad, rel)

    y1 = _project(emb, basis1, comp1, bn)
    p1 = agg1(y1.reshape(n * rel, h), src_p, et_p, dst_p, nrm16, zeros_nh)
    y2 = _combine_project(p1[:, :n], bias1, basis2, comp2, bn)
    p2 = agg2(y2.reshape(n * rel, out_dim), src_p, et_p, dst_p, nrm16, zeros_no)
    return _final(p2[:, :n], bias2, bn)
